# baseline (speedup 1.0000x reference)
import jax
import jax.numpy as jnp
from jax.experimental import pallas as pl

E = 8
GATE_K = 16
K = 256
D_IN = 2048
D_OUT = 2048
T = 256


def _body(x_ref, w0_ref, b0_ref, gw_ref, sel_ref, svh_ref, u_ref, eb_ref,
          r8_ref, o_ref):
    x = x_ref[...]
    g = jnp.dot(x, gw_ref[...], preferred_element_type=jnp.float32)
    n2 = jnp.dot(g * g, sel_ref[...],
                 precision=jax.lax.Precision.HIGHEST)
    logit = jnp.sqrt(n2)
    ii = jax.lax.broadcasted_iota(jnp.int32, (T, E), 1)
    m1 = jnp.max(logit, axis=1, keepdims=True)
    i1 = jnp.min(jnp.where(logit >= m1, ii, E), axis=1, keepdims=True)
    l2 = jnp.where(ii == i1, jnp.float32(-1e30), logit)
    m2 = jnp.max(l2, axis=1, keepdims=True)
    i2 = jnp.min(jnp.where(l2 >= m2, ii, E), axis=1, keepdims=True)
    w1 = 1.0 / (1.0 + jnp.exp(m2 - m1))
    wf = jnp.where(ii == i1, w1, 0.0) + jnp.where(ii == i2, 1.0 - w1, 0.0)

    pret = jnp.dot(x, w0_ref[...], preferred_element_type=jnp.float32)
    pret = pret + b0_ref[...]
    z = jnp.dot(x, svh_ref[...], preferred_element_type=jnp.float32)
    wcols = jnp.dot(wf, r8_ref[...],
                    precision=jax.lax.Precision.HIGHEST)
    zw = (z * wcols).astype(jnp.bfloat16)
    y = jnp.dot(zw, u_ref[...], preferred_element_type=jnp.float32)
    y = y + jnp.dot(wf, eb_ref[...], precision=jax.lax.Precision.HIGHEST)
    o_ref[...] = pret + y


def kernel(hidden_states, W0, b0, gate_W, u, svh, expert_bias):
    Bb, Ss, Dd = hidden_states.shape
    n = Bb * Ss
    bf = jnp.bfloat16
    hs = hidden_states.reshape(n, D_IN).astype(bf)
    W0T = W0.T.astype(bf)
    gWT = gate_W.T.astype(bf)
    SVHT = svh.reshape(E * K, D_IN).T.astype(bf)
    U_all = u.transpose(0, 2, 1).reshape(E * K, D_OUT).astype(bf)
    b0r = b0.reshape(1, D_OUT)
    Sel = (jnp.arange(E * GATE_K)[:, None] // GATE_K
           == jnp.arange(E)[None, :]).astype(jnp.float32)
    R8 = (jnp.arange(E)[:, None]
          == jnp.arange(E * K)[None, :] // K).astype(jnp.float32)

    out = pl.pallas_call(
        _body,
        grid=(n // T,),
        in_specs=[
            pl.BlockSpec((T, D_IN), lambda i: (i, 0)),
            pl.BlockSpec((D_IN, D_OUT), lambda i: (0, 0)),
            pl.BlockSpec((1, D_OUT), lambda i: (0, 0)),
            pl.BlockSpec((D_IN, E * GATE_K), lambda i: (0, 0)),
            pl.BlockSpec((E * GATE_K, E), lambda i: (0, 0)),
            pl.BlockSpec((D_IN, E * K), lambda i: (0, 0)),
            pl.BlockSpec((E * K, D_OUT), lambda i: (0, 0)),
            pl.BlockSpec((E, D_OUT), lambda i: (0, 0)),
            pl.BlockSpec((E, E * K), lambda i: (0, 0)),
        ],
        out_specs=pl.BlockSpec((T, D_OUT), lambda i: (i, 0)),
        out_shape=jax.ShapeDtypeStruct((n, D_OUT), jnp.float32),
    )(hs, W0T, b0r, gWT, Sel, SVHT, U_all, expert_bias, R8)
    return out.reshape(Bb, Ss, D_OUT)


# speedup vs baseline: 1.5129x; 1.5129x over previous
import jax
import jax.numpy as jnp
from jax.experimental import pallas as pl

E = 8
GATE_K = 16
K = 256
D_IN = 2048
D_OUT = 2048
T = 256

_CONTRACT_1_1 = (((1,), (1,)), ((), ()))


def _dot_t(a, b):
    return jax.lax.dot_general(a, b, _CONTRACT_1_1,
                               preferred_element_type=jnp.float32)


def _body(x_ref, w0_ref, b0_ref, gw_ref, sel_ref, svh_ref, u_ref, eb_ref,
          r8_ref, o_ref):
    x = x_ref[...].astype(jnp.bfloat16)
    g = _dot_t(x, gw_ref[...])
    n2 = jnp.dot(g * g, sel_ref[...],
                 precision=jax.lax.Precision.HIGHEST)
    logit = jnp.sqrt(n2)
    ii = jax.lax.broadcasted_iota(jnp.int32, (T, E), 1)
    m1 = jnp.max(logit, axis=1, keepdims=True)
    i1 = jnp.min(jnp.where(logit >= m1, ii, E), axis=1, keepdims=True)
    l2 = jnp.where(ii == i1, jnp.float32(-1e30), logit)
    m2 = jnp.max(l2, axis=1, keepdims=True)
    i2 = jnp.min(jnp.where(l2 >= m2, ii, E), axis=1, keepdims=True)
    w1 = 1.0 / (1.0 + jnp.exp(m2 - m1))
    wf = jnp.where(ii == i1, w1, 0.0) + jnp.where(ii == i2, 1.0 - w1, 0.0)
    wfb = wf.astype(jnp.bfloat16)

    pret = _dot_t(x, w0_ref[...]) + b0_ref[...]
    z = _dot_t(x, svh_ref[...])
    wcols = jnp.dot(wfb, r8_ref[...],
                    preferred_element_type=jnp.float32)
    zw = (z * wcols).astype(jnp.bfloat16)
    y = jnp.dot(wfb, eb_ref[...], preferred_element_type=jnp.float32)
    for e in range(E):
        y = y + _dot_t(zw[:, e * K:(e + 1) * K], u_ref[e])
    o_ref[...] = pret + y


def kernel(hidden_states, W0, b0, gate_W, u, svh, expert_bias):
    Bb, Ss, Dd = hidden_states.shape
    n = Bb * Ss
    bf = jnp.bfloat16
    hs = hidden_states.reshape(n, D_IN)
    W0b = W0.astype(bf)
    gWb = gate_W.astype(bf)
    svh2 = svh.reshape(E * K, D_IN).astype(bf)
    ub = u.astype(bf)
    ebb = expert_bias.astype(bf)
    b0r = b0.reshape(1, D_OUT)
    Sel = (jnp.arange(E * GATE_K)[:, None] // GATE_K
           == jnp.arange(E)[None, :]).astype(jnp.float32)
    R8 = (jnp.arange(E)[:, None]
          == jnp.arange(E * K)[None, :] // K).astype(bf)

    out = pl.pallas_call(
        _body,
        grid=(n // T,),
        in_specs=[
            pl.BlockSpec((T, D_IN), lambda i: (i, 0)),
            pl.BlockSpec((D_OUT, D_IN), lambda i: (0, 0)),
            pl.BlockSpec((1, D_OUT), lambda i: (0, 0)),
            pl.BlockSpec((E * GATE_K, D_IN), lambda i: (0, 0)),
            pl.BlockSpec((E * GATE_K, E), lambda i: (0, 0)),
            pl.BlockSpec((E * K, D_IN), lambda i: (0, 0)),
            pl.BlockSpec((E, D_OUT, K), lambda i: (0, 0, 0)),
            pl.BlockSpec((E, D_OUT), lambda i: (0, 0)),
            pl.BlockSpec((E, E * K), lambda i: (0, 0)),
        ],
        out_specs=pl.BlockSpec((T, D_OUT), lambda i: (i, 0)),
        out_shape=jax.ShapeDtypeStruct((n, D_OUT), jnp.float32),
    )(hs, W0b, b0r, gWb, Sel, svh2, ub, ebb, R8)
    return out.reshape(Bb, Ss, D_OUT)


# speedup vs baseline: 1.5315x; 1.0123x over previous
import jax
import jax.numpy as jnp
from jax.experimental import pallas as pl

E = 8
GATE_K = 16
K = 256
D_IN = 2048
D_OUT = 2048
T = 512

_CONTRACT_1_1 = (((1,), (1,)), ((), ()))


def _dot_t(a, b):
    return jax.lax.dot_general(a, b, _CONTRACT_1_1,
                               preferred_element_type=jnp.float32)


def _body(x_ref, w0_ref, b0_ref, gw_ref, sel_ref, svh_ref, u_ref, eb_ref,
          r8_ref, o_ref):
    x = x_ref[...].astype(jnp.bfloat16)
    g = _dot_t(x, gw_ref[...])
    n2 = jnp.dot(g * g, sel_ref[...],
                 precision=jax.lax.Precision.HIGHEST)
    logit = jnp.sqrt(n2)
    ii = jax.lax.broadcasted_iota(jnp.int32, (T, E), 1)
    m1 = jnp.max(logit, axis=1, keepdims=True)
    i1 = jnp.min(jnp.where(logit >= m1, ii, E), axis=1, keepdims=True)
    l2 = jnp.where(ii == i1, jnp.float32(-1e30), logit)
    m2 = jnp.max(l2, axis=1, keepdims=True)
    i2 = jnp.min(jnp.where(l2 >= m2, ii, E), axis=1, keepdims=True)
    w1 = 1.0 / (1.0 + jnp.exp(m2 - m1))
    wf = jnp.where(ii == i1, w1, 0.0) + jnp.where(ii == i2, 1.0 - w1, 0.0)
    wfb = wf.astype(jnp.bfloat16)

    pret = _dot_t(x, w0_ref[...]) + b0_ref[...]
    z = _dot_t(x, svh_ref[...])
    wcols = jnp.dot(wfb, r8_ref[...],
                    preferred_element_type=jnp.float32)
    zw = (z * wcols).astype(jnp.bfloat16)
    y = jnp.dot(wfb, eb_ref[...], preferred_element_type=jnp.float32)
    for e in range(E):
        y = y + _dot_t(zw[:, e * K:(e + 1) * K], u_ref[e])
    o_ref[...] = pret + y


def kernel(hidden_states, W0, b0, gate_W, u, svh, expert_bias):
    Bb, Ss, Dd = hidden_states.shape
    n = Bb * Ss
    bf = jnp.bfloat16
    hs = hidden_states.reshape(n, D_IN)
    W0b = W0.astype(bf)
    gWb = gate_W.astype(bf)
    svh2 = svh.reshape(E * K, D_IN).astype(bf)
    ub = u.astype(bf)
    ebb = expert_bias.astype(bf)
    b0r = b0.reshape(1, D_OUT)
    Sel = (jnp.arange(E * GATE_K)[:, None] // GATE_K
           == jnp.arange(E)[None, :]).astype(jnp.float32)
    R8 = (jnp.arange(E)[:, None]
          == jnp.arange(E * K)[None, :] // K).astype(bf)

    out = pl.pallas_call(
        _body,
        grid=(n // T,),
        in_specs=[
            pl.BlockSpec((T, D_IN), lambda i: (i, 0)),
            pl.BlockSpec((D_OUT, D_IN), lambda i: (0, 0)),
            pl.BlockSpec((1, D_OUT), lambda i: (0, 0)),
            pl.BlockSpec((E * GATE_K, D_IN), lambda i: (0, 0)),
            pl.BlockSpec((E * GATE_K, E), lambda i: (0, 0)),
            pl.BlockSpec((E * K, D_IN), lambda i: (0, 0)),
            pl.BlockSpec((E, D_OUT, K), lambda i: (0, 0, 0)),
            pl.BlockSpec((E, D_OUT), lambda i: (0, 0)),
            pl.BlockSpec((E, E * K), lambda i: (0, 0)),
        ],
        out_specs=pl.BlockSpec((T, D_OUT), lambda i: (i, 0)),
        out_shape=jax.ShapeDtypeStruct((n, D_OUT), jnp.float32),
    )(hs, W0b, b0r, gWb, Sel, svh2, ub, ebb, R8)
    return out.reshape(Bb, Ss, D_OUT)
